# baseline (speedup 1.0000x reference)
import functools

import jax
import jax.numpy as jnp
from jax import lax
from jax.experimental import pallas as pl
from jax.experimental.pallas import tpu as pltpu
from jax.experimental.pallas import tpu_sc as plsc

N = 10000
D = 128
E = 320000
NC = 2
NS = 16
NW = NC * NS
EPT = E // NW
CH = 100
NCHUNK = EPT // CH
BLK = 20
NBLK = NCHUNK // BLK
RPT = N // NS

_SHUF = []
for _c in range(D // 32):
    for _k in range(32):
        _SHUF.append(32 * _c + (2 * _k if _k < 16 else 2 * (_k - 16) + 1))
_INV = [0] * D
for _p, _s in enumerate(_SHUF):
    _INV[_s] = _p

_MESH = plsc.VectorSubcoreMesh(
    core_axis_name="c", subcore_axis_name="s", num_cores=NC, num_subcores=NS
)
_SC_PARAMS = pltpu.CompilerParams(use_tc_tiling_on_sc=False)
_SC_PARAMS_NL = pltpu.CompilerParams(use_tc_tiling_on_sc=False,
                                     needs_layout_passes=False)


def _deg_body(dst_hbm, ones_hbm, zero_hbm, out_hbm, dst_v, ones_v, dacc, dsem):
    c = lax.axis_index("c")
    s = lax.axis_index("s")
    wid = c * NS + s
    pltpu.sync_copy(dst_hbm.at[wid], dst_v)
    pltpu.sync_copy(ones_hbm, ones_v)

    @pl.when(s == 0)
    def _zero():
        pltpu.sync_copy(zero_hbm, dacc)

    plsc.subcore_barrier()

    for b in range(NBLK):
        for j in range(BLK):
            pltpu.async_copy(ones_v, dacc.at[dst_v.at[b * BLK + j]], dsem,
                             add=True)
        for j in range(BLK):
            pltpu.make_async_copy(ones_v, dacc.at[dst_v.at[b * BLK + j]],
                                  dsem).wait()

    plsc.subcore_barrier()

    @pl.when(s == 0)
    def _writeback():
        pltpu.sync_copy(dacc, out_hbm.at[c, 0])


_deg_call = functools.partial(
    pl.kernel,
    out_type=jax.ShapeDtypeStruct((NC, 1, N), jnp.float32),
    mesh=_MESH,
    compiler_params=_SC_PARAMS,
    scratch_types=[
        pltpu.VMEM((NCHUNK, CH), jnp.int32),
        pltpu.VMEM((CH,), jnp.float32),
        pltpu.VMEM_SHARED((N,), jnp.float32),
        pltpu.SemaphoreType.DMA,
    ],
)(_deg_body)


def _scat_body(g_hbm, src_hbm, dst_hbm, zero_hbm, out_hbm,
               sblk, dblk, bb0, bb1, fb0, fb1, acc,
               gs0, gs1, ss0, ss1):
    c = lax.axis_index("c")
    s = lax.axis_index("s")
    wid = c * NS + s
    base = s * RPT
    pltpu.sync_copy(zero_hbm, acc.at[pl.ds(base, RPT)])
    plsc.subcore_barrier()

    def start_g(j, bb, sem):
        pltpu.async_copy(g_hbm.at[sblk.at[j]], bb, sem)

    def wait_g(j, bb, sem):
        pltpu.make_async_copy(g_hbm.at[sblk.at[j]], bb, sem).wait()

    def start_s(j, fb, sem):
        pltpu.async_copy(fb, acc.at[dblk.at[j]], sem, add=True)

    def wait_s(j, fb, sem):
        pltpu.make_async_copy(fb, acc.at[dblk.at[j]], sem).wait()

    def conv(bb, fb):
        def cb(i, carry):
            for rr in range(4):
                r = i * 4 + rr
                for cc in range(D // 32):
                    raw = plsc.bitcast(bb[r, pl.ds(cc * 32, 32)], jnp.int32)
                    lo = plsc.bitcast(lax.shift_left(raw, 16), jnp.float32)
                    hi = plsc.bitcast(
                        jnp.bitwise_and(raw, jnp.int32(-65536)), jnp.float32)
                    fb[r, pl.ds(cc * 32, 16)] = lo
                    fb[r, pl.ds(cc * 32 + 16, 16)] = hi
            return carry
        lax.fori_loop(0, CH // 4, cb, 0)

    def block_body(b, carry):
        off = b * BLK
        pltpu.sync_copy(src_hbm.at[wid, pl.ds(off, BLK)], sblk)
        pltpu.sync_copy(dst_hbm.at[wid, pl.ds(off, BLK)], dblk)
        start_g(0, bb0, gs0)
        start_g(1, bb1, gs1)
        wait_g(0, bb0, gs0)
        conv(bb0, fb0)
        start_s(0, fb0, ss0)
        start_g(2, bb0, gs0)
        wait_g(1, bb1, gs1)
        conv(bb1, fb1)
        start_s(1, fb1, ss1)
        start_g(3, bb1, gs1)

        def inner(i, icarry):
            wait_g(2 * i, bb0, gs0)
            wait_s(2 * i - 2, fb0, ss0)
            conv(bb0, fb0)
            start_s(2 * i, fb0, ss0)
            start_g(2 * i + 2, bb0, gs0)
            wait_g(2 * i + 1, bb1, gs1)
            wait_s(2 * i - 1, fb1, ss1)
            conv(bb1, fb1)
            start_s(2 * i + 1, fb1, ss1)
            start_g(2 * i + 3, bb1, gs1)
            return icarry

        lax.fori_loop(1, BLK // 2 - 1, inner, 0)
        wait_g(BLK - 2, bb0, gs0)
        wait_s(BLK - 4, fb0, ss0)
        conv(bb0, fb0)
        start_s(BLK - 2, fb0, ss0)
        wait_g(BLK - 1, bb1, gs1)
        wait_s(BLK - 3, fb1, ss1)
        conv(bb1, fb1)
        start_s(BLK - 1, fb1, ss1)
        wait_s(BLK - 2, fb0, ss0)
        wait_s(BLK - 1, fb1, ss1)
        return carry

    lax.fori_loop(0, NBLK, block_body, 0)

    plsc.subcore_barrier()
    pltpu.sync_copy(acc.at[pl.ds(base, RPT)], out_hbm.at[c, pl.ds(base, RPT)])


_scat_call = functools.partial(
    pl.kernel,
    out_type=jax.ShapeDtypeStruct((NC, N, D), jnp.float32),
    mesh=_MESH,
    compiler_params=_SC_PARAMS_NL,
    scratch_types=[
        pltpu.VMEM((BLK, CH), jnp.int32),
        pltpu.VMEM((BLK, CH), jnp.int32),
        pltpu.VMEM((CH, D), jnp.bfloat16),
        pltpu.VMEM((CH, D), jnp.bfloat16),
        pltpu.VMEM((CH, D), jnp.float32),
        pltpu.VMEM((CH, D), jnp.float32),
        pltpu.VMEM_SHARED((N, D), jnp.float32),
        pltpu.SemaphoreType.DMA,
        pltpu.SemaphoreType.DMA,
        pltpu.SemaphoreType.DMA,
        pltpu.SemaphoreType.DMA,
    ],
)(_scat_body)


def _dis(degt_ref):
    return lax.rsqrt(degt_ref[:, 0:1] + degt_ref[:, 1:2] + 1.0)


def _dense_a1_body(x_ref, w1_ref, w1i_ref, h_ref, hi_ref):
    h_ref[...] = jnp.dot(x_ref[...], w1_ref[...],
                         preferred_element_type=jnp.float32)
    hi_ref[...] = jnp.dot(x_ref[...], w1i_ref[...],
                          preferred_element_type=jnp.float32)


def _dense_a2_body(h_ref, hi_ref, degt_ref, g1_ref, g1b_ref):
    dis = _dis(degt_ref)
    g1_ref[...] = h_ref[...] * dis
    g1b_ref[...] = (hi_ref[...] * dis).astype(jnp.bfloat16)


def _dense_b_body(p_ref, g1_ref, degt_ref, b1_ref, w2_ref, w2i_ref,
                  g2_ref, g2b_ref):
    dis = _dis(degt_ref)
    z = (p_ref[0] + p_ref[1] + g1_ref[...]) * dis + b1_ref[...]
    z = jnp.maximum(z, 0.0)
    g2_ref[...] = jnp.dot(z, w2_ref[...], preferred_element_type=jnp.float32) * dis
    g2b_ref[...] = (jnp.dot(z, w2i_ref[...],
                            preferred_element_type=jnp.float32) * dis
                    ).astype(jnp.bfloat16)


def _dense_c_body(p_ref, g2_ref, degt_ref, b2_ref, gam_ref, bet_ref, o_ref):
    h = (p_ref[0] + p_ref[1] + g2_ref[...]) * _dis(degt_ref) + b2_ref[...]
    mu = jnp.mean(h, axis=-1, keepdims=True)
    d = h - mu
    var = jnp.mean(d * d, axis=-1, keepdims=True)
    o_ref[...] = d * lax.rsqrt(var + 1e-5) * gam_ref[...] + bet_ref[...]


def _out(dtype=jnp.float32):
    return jax.ShapeDtypeStruct((N, D), dtype)


_dense_a1 = pl.pallas_call(_dense_a1_body, out_shape=(_out(), _out()))
_dense_a2 = pl.pallas_call(
    _dense_a2_body, out_shape=(_out(), _out(jnp.bfloat16)))
_dense_b = pl.pallas_call(
    _dense_b_body, out_shape=(_out(), _out(jnp.bfloat16)))
_dense_c = pl.pallas_call(_dense_c_body, out_shape=_out())


def kernel(x, edge_index, W1, b1, W2, b2, gamma, beta):
    ei = edge_index.astype(jnp.int32)
    src = ei[0].reshape(NW, NCHUNK, CH)
    dst = ei[1].reshape(NW, NCHUNK, CH)
    zeros2d = jnp.zeros((RPT, D), jnp.float32)
    zeros1d = jnp.zeros((N,), jnp.float32)
    ones_ch = jnp.ones((CH,), jnp.float32)
    inv = jnp.asarray(_INV, dtype=jnp.int32)
    W1i = jnp.take(W1, inv, axis=1)
    W2i = jnp.take(W2, inv, axis=1)

    degp = _deg_call(dst, ones_ch, zeros1d)
    h1, h1i = _dense_a1(x, W1, W1i)
    degt = degp[:, 0, :].T

    g1, g1b = _dense_a2(h1, h1i, degt)
    p1 = _scat_call(g1b, src, dst, zeros2d)
    g2, g2b = _dense_b(p1, g1, degt, b1.reshape(1, D), W2, W2i)
    p2 = _scat_call(g2b, src, dst, zeros2d)
    return _dense_c(p2, g2, degt, b2.reshape(1, D),
                    gamma.reshape(1, D), beta.reshape(1, D))


# speedup vs baseline: 1.8936x; 1.8936x over previous
import functools

import jax
import jax.numpy as jnp
from jax import lax
from jax.experimental import pallas as pl
from jax.experimental.pallas import tpu as pltpu
from jax.experimental.pallas import tpu_sc as plsc

N = 10000
D = 128
E = 320000
NC = 2
NS = 16
NW = NC * NS
EPT = E // NW
CH = 80
NCHUNK = EPT // CH
NBUF = 3
BLK = 25
NBLK = NCHUNK // BLK
RPT = N // NS

_MESH = plsc.VectorSubcoreMesh(
    core_axis_name="c", subcore_axis_name="s", num_cores=NC, num_subcores=NS
)
_SC_PARAMS = pltpu.CompilerParams(use_tc_tiling_on_sc=False)


def _deg_body(dst_hbm, ones_hbm, zero_hbm, out_hbm, dst_v, ones_v, dacc, dsem):
    c = lax.axis_index("c")
    s = lax.axis_index("s")
    wid = c * NS + s
    pltpu.sync_copy(dst_hbm.at[wid], dst_v)
    pltpu.sync_copy(ones_hbm, ones_v)

    @pl.when(s == 0)
    def _zero():
        pltpu.sync_copy(zero_hbm, dacc)

    plsc.subcore_barrier()

    for b in range(NBLK):
        for j in range(BLK):
            pltpu.async_copy(ones_v, dacc.at[dst_v.at[b * BLK + j]], dsem,
                             add=True)
        for j in range(BLK):
            pltpu.make_async_copy(ones_v, dacc.at[dst_v.at[b * BLK + j]],
                                  dsem).wait()

    plsc.subcore_barrier()

    @pl.when(s == 0)
    def _writeback():
        pltpu.sync_copy(dacc, out_hbm.at[c, 0])


_deg_call = functools.partial(
    pl.kernel,
    out_type=jax.ShapeDtypeStruct((NC, 1, N), jnp.float32),
    mesh=_MESH,
    compiler_params=_SC_PARAMS,
    scratch_types=[
        pltpu.VMEM((NCHUNK, CH), jnp.int32),
        pltpu.VMEM((CH,), jnp.float32),
        pltpu.VMEM_SHARED((N,), jnp.float32),
        pltpu.SemaphoreType.DMA,
    ],
)(_deg_body)


def _scat_body(g_hbm, src_hbm, dst_hbm, zero_hbm, out_hbm,
               sA, dA, sB, dB, b0, b1, b2, acc,
               g0, g1, g2, s0, s1, s2, isem):
    c = lax.axis_index("c")
    s = lax.axis_index("s")
    wid = c * NS + s
    base = s * RPT
    pltpu.sync_copy(zero_hbm, acc.at[pl.ds(base, RPT)])
    pltpu.sync_copy(src_hbm.at[wid, pl.ds(0, BLK)], sA)
    pltpu.sync_copy(dst_hbm.at[wid, pl.ds(0, BLK)], dA)
    plsc.subcore_barrier()

    bufs = (b0, b1, b2)
    gsems = (g0, g1, g2)
    ssems = (s0, s1, s2)
    sblk = (sA, sB)
    dblk = (dA, dB)

    def srow(ci):
        return sblk[(ci // BLK) % 2].at[ci % BLK]

    def drow(ci):
        return dblk[(ci // BLK) % 2].at[ci % BLK]

    def start_g(ci):
        k = ci % NBUF
        pltpu.async_copy(g_hbm.at[srow(ci)], bufs[k], gsems[k])

    def wait_g(ci):
        k = ci % NBUF
        pltpu.make_async_copy(g_hbm.at[srow(ci)], bufs[k], gsems[k]).wait()

    def start_s(ci):
        k = ci % NBUF
        pltpu.async_copy(bufs[k], acc.at[drow(ci)], ssems[k], add=True)

    def wait_s(ci):
        k = ci % NBUF
        pltpu.make_async_copy(bufs[k], acc.at[drow(ci)], ssems[k]).wait()

    for ci in range(NCHUNK):
        blkid = ci // BLK
        if ci >= NBUF:
            wait_s(ci - NBUF)
        start_g(ci)
        if ci >= 1:
            wait_g(ci - 1)
            start_s(ci - 1)
        if ci % BLK == NBUF and blkid + 1 < NBLK:
            nxt = (blkid + 1) % 2
            off = (blkid + 1) * BLK
            pltpu.async_copy(src_hbm.at[wid, pl.ds(off, BLK)], sblk[nxt], isem)
            pltpu.async_copy(dst_hbm.at[wid, pl.ds(off, BLK)], dblk[nxt], isem)
        if ci % BLK == BLK - 1 and blkid + 1 < NBLK:
            nxt = (blkid + 1) % 2
            off = (blkid + 1) * BLK
            pltpu.make_async_copy(src_hbm.at[wid, pl.ds(off, BLK)], sblk[nxt],
                                  isem).wait()
            pltpu.make_async_copy(dst_hbm.at[wid, pl.ds(off, BLK)], dblk[nxt],
                                  isem).wait()

    wait_g(NCHUNK - 1)
    start_s(NCHUNK - 1)
    for ci in range(NCHUNK - NBUF, NCHUNK):
        wait_s(ci)

    plsc.subcore_barrier()
    pltpu.sync_copy(acc.at[pl.ds(base, RPT)], out_hbm.at[c, pl.ds(base, RPT)])


_scat_call = functools.partial(
    pl.kernel,
    out_type=jax.ShapeDtypeStruct((NC, N, D), jnp.float32),
    mesh=_MESH,
    compiler_params=_SC_PARAMS,
    scratch_types=[
        pltpu.VMEM((BLK, CH), jnp.int32),
        pltpu.VMEM((BLK, CH), jnp.int32),
        pltpu.VMEM((BLK, CH), jnp.int32),
        pltpu.VMEM((BLK, CH), jnp.int32),
        pltpu.VMEM((CH, D), jnp.float32),
        pltpu.VMEM((CH, D), jnp.float32),
        pltpu.VMEM((CH, D), jnp.float32),
        pltpu.VMEM_SHARED((N, D), jnp.float32),
        pltpu.SemaphoreType.DMA,
        pltpu.SemaphoreType.DMA,
        pltpu.SemaphoreType.DMA,
        pltpu.SemaphoreType.DMA,
        pltpu.SemaphoreType.DMA,
        pltpu.SemaphoreType.DMA,
        pltpu.SemaphoreType.DMA,
    ],
)(_scat_body)


def _dis(degt_ref):
    return lax.rsqrt(degt_ref[:, 0:1] + degt_ref[:, 1:2] + 1.0)


def _dense_a1_body(x_ref, w1_ref, h_ref):
    h_ref[...] = jnp.dot(x_ref[...], w1_ref[...],
                         preferred_element_type=jnp.float32)


def _dense_a2_body(h_ref, degt_ref, g1_ref):
    g1_ref[...] = h_ref[...] * _dis(degt_ref)


def _dense_b_body(p_ref, g1_ref, degt_ref, b1_ref, w2_ref, g2_ref):
    dis = _dis(degt_ref)
    z = (p_ref[0] + p_ref[1] + g1_ref[...]) * dis + b1_ref[...]
    z = jnp.maximum(z, 0.0)
    g2_ref[...] = jnp.dot(z, w2_ref[...], preferred_element_type=jnp.float32) * dis


def _dense_c_body(p_ref, g2_ref, degt_ref, b2_ref, gam_ref, bet_ref, o_ref):
    h = (p_ref[0] + p_ref[1] + g2_ref[...]) * _dis(degt_ref) + b2_ref[...]
    mu = jnp.mean(h, axis=-1, keepdims=True)
    d = h - mu
    var = jnp.mean(d * d, axis=-1, keepdims=True)
    o_ref[...] = d * lax.rsqrt(var + 1e-5) * gam_ref[...] + bet_ref[...]


def _tc_call(body):
    return pl.pallas_call(
        body,
        out_shape=jax.ShapeDtypeStruct((N, D), jnp.float32),
    )


_dense_a1 = _tc_call(_dense_a1_body)
_dense_a2 = _tc_call(_dense_a2_body)
_dense_b = _tc_call(_dense_b_body)
_dense_c = _tc_call(_dense_c_body)


def kernel(x, edge_index, W1, b1, W2, b2, gamma, beta):
    ei = edge_index.astype(jnp.int32)
    src = ei[0].reshape(NW, NCHUNK, CH)
    dst = ei[1].reshape(NW, NCHUNK, CH)
    zeros2d = jnp.zeros((RPT, D), jnp.float32)
    zeros1d = jnp.zeros((N,), jnp.float32)
    ones_ch = jnp.ones((CH,), jnp.float32)

    degp = _deg_call(dst, ones_ch, zeros1d)
    h1 = _dense_a1(x, W1)
    degt = degp[:, 0, :].T

    g1 = _dense_a2(h1, degt)
    p1 = _scat_call(g1, src, dst, zeros2d)
    g2 = _dense_b(p1, g1, degt, b1.reshape(1, D), W2)
    p2 = _scat_call(g2, src, dst, zeros2d)
    return _dense_c(p2, g2, degt, b2.reshape(1, D),
                    gamma.reshape(1, D), beta.reshape(1, D))
